# Initial kernel scaffold; baseline (speedup 1.0000x reference)
import jax
import jax.numpy as jnp
from jax.experimental import pallas as pl


def kernel(coords, edge_index, edge_index_2rd, edx_1st, edx_2nd, E, is_source, edge_rep, params):
    raise NotImplementedError("write your pallas kernel here")


# speedup vs baseline: 1.1178x; 1.1178x over previous
import functools
from math import pi as PI

import jax
import jax.numpy as jnp
from jax import lax
from jax.experimental import pallas as pl
from jax.experimental.pallas import tpu as pltpu

H = 128
NG = 12
MSGW = 144


def _theta_body(cross_ref, dot_ref, out_ref):
    c = cross_ref[...]
    d = dot_ref[...]
    th = jnp.arctan2(jnp.abs(c), d)
    flag = jnp.where(c > 0, 1.0, -1.0)
    out_ref[...] = th * flag


def _geo_body(th_ref, d2_ref, wd1_ref, bd1_ref, wd2_ref, bd2_ref,
              wt1_ref, bt1_ref, wt2_ref, bt2_ref, wg_ref, bg_ref,
              gp0_ref, gp1_ref, gp2_ref):
    th = th_ref[...]
    lane = lax.broadcasted_iota(jnp.int32, (1, H), 1)
    delta = 2.0 * PI / (NG - 1)
    offs = -PI + lane.astype(jnp.float32) * delta
    coeff = -0.5 / (delta * delta)
    texp = jnp.exp(coeff * (th - offs) ** 2)
    texp = jnp.where(lane < NG, texp, 0.0)
    ge2 = jax.nn.relu(jnp.dot(texp, wt1_ref[...]) + bt1_ref[...])
    ge2 = jax.nn.relu(jnp.dot(ge2, wt2_ref[...]) + bt2_ref[...])

    d2 = d2_ref[...]
    invd = jnp.where(d2 == 0.0, 1e10, lax.rsqrt(d2))
    ge1 = jax.nn.relu(invd * wd1_ref[...] + bd1_ref[...])
    ge1 = jax.nn.relu(jnp.dot(ge1, wd2_ref[...]) + bd2_ref[...])

    for b, ref in enumerate((gp0_ref, gp1_ref, gp2_ref)):
        ref[...] = (jnp.dot(ge1, wg_ref[b, :H, :])
                    + jnp.dot(ge2, wg_ref[b, H:, :]) + bg_ref[b])


def _blk_a_body(nfi_ref, nfj_ref, gp_ref, wi_ref, wj_ref, w2_ref, b2_ref,
                w3_ref, b3_ref, w4_ref, b4_ref, att_ref, alpha_ref):
    h = jax.nn.relu(jnp.dot(nfi_ref[...], wi_ref[...])
                    + jnp.dot(nfj_ref[...], wj_ref[...]) + gp_ref[...])
    h = jax.nn.relu(jnp.dot(h, w2_ref[...]) + b2_ref[...])
    h = jax.nn.relu(jnp.dot(h, w3_ref[...]) + b3_ref[...])
    h = jax.nn.relu(jnp.dot(h, w4_ref[...]) + b4_ref[...])
    t = h * att_ref[...]
    t = jnp.where(t >= 0.0, t, 0.01 * t)
    alpha_ref[...] = jnp.sum(t, axis=1, keepdims=True)


def _blk_b_body(nfj_ref, alpha_ref, amaxe_ref, out_ref):
    ex = jnp.exp(alpha_ref[...] - amaxe_ref[...])
    te = ex.shape[0]
    out_ref[...] = jnp.concatenate(
        [nfj_ref[...] * ex, ex, jnp.zeros((te, MSGW - H - 1), jnp.float32)],
        axis=1)


def _fin_body(acc_ref, nf_ref, src_ref, out_ref):
    acc = acc_ref[...]
    numer = acc[:, :H]
    denom = acc[:, H:H + 1]
    out_ref[...] = jnp.where(src_ref[...] > 0.0, nf_ref[...],
                             numer / (denom + 1e-16))


def _theta_call(cross2d, dot2d):
    nr = cross2d.shape[0]
    br = 1000
    spec = pl.BlockSpec((br, H), lambda i: (i, 0))
    return pl.pallas_call(
        _theta_body,
        grid=(nr // br,),
        in_specs=[spec, spec],
        out_specs=spec,
        out_shape=jax.ShapeDtypeStruct((nr, H), jnp.float32),
    )(cross2d, dot2d)


def _geo_call(thetas1, d2, wp):
    ne = thetas1.shape[0]
    te = 2000
    col = pl.BlockSpec((te, 1), lambda i: (i, 0))
    out = pl.BlockSpec((te, H), lambda i: (i, 0))
    full = lambda a: pl.BlockSpec(a.shape, lambda i: (0,) * a.ndim)
    w_list = [wp['wd1'], wp['bd1'], wp['wd2'], wp['bd2'],
              wp['wt1'], wp['bt1'], wp['wt2'], wp['bt2'],
              wp['wg'], wp['bg']]
    return pl.pallas_call(
        _geo_body,
        grid=(ne // te,),
        in_specs=[col, col] + [full(w) for w in w_list],
        out_specs=[out, out, out],
        out_shape=[jax.ShapeDtypeStruct((ne, H), jnp.float32)] * 3,
    )(thetas1, d2, *w_list)


def _blk_a_call(nfi_g, nfj_g, gp, bw):
    ne = nfi_g.shape[0]
    te = 2000
    mat = pl.BlockSpec((te, H), lambda i: (i, 0))
    col = pl.BlockSpec((te, 1), lambda i: (i, 0))
    full = lambda a: pl.BlockSpec(a.shape, lambda i: (0,) * a.ndim)
    w_list = [bw['wi'], bw['wj'], bw['w2'], bw['b2'],
              bw['w3'], bw['b3'], bw['w4'], bw['b4'], bw['att']]
    return pl.pallas_call(
        _blk_a_body,
        grid=(ne // te,),
        in_specs=[mat, mat, mat] + [full(w) for w in w_list],
        out_specs=col,
        out_shape=jax.ShapeDtypeStruct((ne, 1), jnp.float32),
    )(nfi_g, nfj_g, gp, *w_list)


def _blk_b_call(nfj_g, alpha, amax_e):
    ne = nfj_g.shape[0]
    te = 2000
    mat = pl.BlockSpec((te, H), lambda i: (i, 0))
    col = pl.BlockSpec((te, 1), lambda i: (i, 0))
    out = pl.BlockSpec((te, MSGW), lambda i: (i, 0))
    return pl.pallas_call(
        _blk_b_body,
        grid=(ne // te,),
        in_specs=[mat, col, col],
        out_specs=out,
        out_shape=jax.ShapeDtypeStruct((ne, MSGW), jnp.float32),
    )(nfj_g, alpha, amax_e)


def _fin_call(acc, nf, srcmask):
    n = nf.shape[0]
    tn = 2000
    return pl.pallas_call(
        _fin_body,
        grid=(n // tn,),
        in_specs=[pl.BlockSpec((tn, MSGW), lambda i: (i, 0)),
                  pl.BlockSpec((tn, H), lambda i: (i, 0)),
                  pl.BlockSpec((tn, 1), lambda i: (i, 0))],
        out_specs=pl.BlockSpec((tn, H), lambda i: (i, 0)),
        out_shape=jax.ShapeDtypeStruct((n, H), jnp.float32),
    )(acc, nf, srcmask)


def kernel(coords, edge_index, edge_index_2rd, edx_1st, edx_2nd, E,
           is_source, edge_rep, params):
    n = coords.shape[0]
    ne = edge_index.shape[1]
    nt = edge_index_2rd.shape[1]

    (wd1, bd1), (wd2, bd2) = params['dist']
    (wt1, bt1), (wt2, bt2) = params['theta']
    wt1p = jnp.zeros((H, H), jnp.float32).at[:NG, :].set(wt1.T)
    wp = dict(wd1=wd1.T, bd1=bd1[None, :], wd2=wd2.T, bd2=bd2[None, :],
              wt1=wt1p, bt1=bt1[None, :], wt2=wt2.T, bt2=bt2[None, :])
    wgs, bgs, bws = [], [], []
    for blk in params['blocks']:
        (w1, b1) = blk['lins'][0]
        wgs.append(w1[:, 2 * H:].T)
        bgs.append(b1[None, :])
        bw = dict(wi=w1[:, :H].T, wj=w1[:, H:2 * H].T, att=blk['att'])
        for li, (w, b) in enumerate(blk['lins'][1:], start=2):
            bw[f'w{li}'] = w.T
            bw[f'b{li}'] = b[None, :]
        bws.append(bw)
    wp['wg'] = jnp.stack(wgs)
    wp['bg'] = jnp.stack(bgs)

    i2, j2, k2 = edge_index_2rd
    g = coords
    v1 = g[j2] - g[i2]
    v2 = g[k2] - g[j2]
    cross = v1[:, 0] * v2[:, 1] - v1[:, 1] * v2[:, 0]
    dot = (v1 * v2).sum(axis=-1)
    nr = nt // H
    vals = _theta_call(cross.reshape(nr, H), dot.reshape(nr, H)).reshape(nt)
    seg_min = jax.ops.segment_min(vals, edx_2nd, num_segments=ne)
    thetas1 = jnp.where(jnp.isfinite(seg_min), seg_min, 0.0)

    de = g[edge_index[0]] - g[edge_index[1]]
    d2 = (de * de).sum(axis=-1)

    gp0, gp1, gp2 = _geo_call(thetas1[:, None], d2[:, None], wp)

    srcmask = is_source.astype(jnp.float32)[:, None]
    jidx = edge_index[0]
    iidx = edge_index[1]
    nf = E
    for b, gp in enumerate((gp0, gp1, gp2)):
        nfi_g = nf[iidx]
        nfj_g = nf[jidx]
        alpha = _blk_a_call(nfi_g, nfj_g, gp, bws[b])
        amax = jax.ops.segment_max(alpha[:, 0], iidx, num_segments=n)
        amax = jnp.where(jnp.isfinite(amax), amax, 0.0)
        msg = _blk_b_call(nfj_g, alpha, amax[iidx][:, None])
        acc = jax.ops.segment_sum(msg, iidx, num_segments=n)
        nf = _fin_call(acc, nf, srcmask)
    return nf


# speedup vs baseline: 1.2403x; 1.1096x over previous
import functools
from math import pi as PI

import jax
import jax.numpy as jnp
from jax import lax
from jax.experimental import pallas as pl
from jax.experimental.pallas import tpu as pltpu
from jax.experimental.pallas import tpu_sc as plsc

H = 128
NG = 12
MSGW = 144
NW = 32
GCH = 128
_USE_SC_SEGMIN = False

_SC_MESH = dict(core_axis_name="c", subcore_axis_name="s")


def _wid():
    return lax.axis_index("s") * 2 + lax.axis_index("c")


def _rr_loop(nc_total, wid, body):
    ncp = -(-nc_total // NW) * NW

    def f(c, _):
        body(c * NW + wid)
        return 0
    lax.fori_loop(0, ncp // NW, f, 0)


def _scan_chunk(vals_v, idx_v, mloc, seg_base, nseg, ch, negate):
    def one_pass(pending, v, ec):
        m = plsc.load_gather(mloc, [ec])
        want = pending & (v < m)
        plsc.store_scatter(mloc, [ec], v, mask=want)
        m2 = plsc.load_gather(mloc, [ec])
        return pending & (m2 > v)

    def vreg_body(q, _):
        v = vals_v[pl.ds(q * 16, 16)]
        if negate:
            v = -v
        e = idx_v[pl.ds(q * 16, 16)] - seg_base
        inb = (e >= 0) & (e < nseg)
        ec = jnp.where(inb, e, 0)
        pending = inb
        for _ in range(8):
            pending = one_pass(pending, v, ec)
        return 0
    lax.fori_loop(0, ch // 16, vreg_body, 0)


def _sc_gather2(nf, iidx, jidx):
    ne = iidx.shape[0]
    nchunks = ne // GCH

    @functools.partial(
        pl.kernel,
        out_type=[jax.ShapeDtypeStruct((ne, H), jnp.float32)] * 2,
        mesh=plsc.VectorSubcoreMesh(**_SC_MESH),
        scratch_types=[pltpu.VMEM((GCH,), jnp.int32),
                       pltpu.VMEM((GCH, H), jnp.float32),
                       pltpu.SemaphoreType.DMA],
    )
    def k(nf_hbm, i_hbm, j_hbm, oi_hbm, oj_hbm, idx_v, rows_v, sem):
        wid = _wid()

        def one(g):
            base = jnp.minimum(g, nchunks - 1) * GCH
            for src, dst in ((i_hbm, oi_hbm), (j_hbm, oj_hbm)):
                pltpu.sync_copy(src.at[pl.ds(base, GCH)], idx_v)
                pltpu.async_copy(nf_hbm.at[idx_v], rows_v, sem).wait()
                pltpu.sync_copy(rows_v, dst.at[pl.ds(base, GCH)])
        _rr_loop(nchunks, wid, one)

    return k(nf, iidx, jidx)


W2 = 2 * H
HALF = 5120
NROW = 5248


def _sc_scatter_msg(msg, iidx, zeros):
    ne = msg.shape[0]
    nchunks = ne // GCH
    ncp = -(-nchunks // 16) * 16
    rpt = NROW // 16

    @functools.partial(
        pl.kernel,
        out_type=jax.ShapeDtypeStruct((2, NROW, W2), jnp.float32),
        mesh=plsc.VectorSubcoreMesh(**_SC_MESH),
        scratch_types=[pltpu.VMEM((GCH,), jnp.int32),
                       pltpu.VMEM((GCH, W2), jnp.float32),
                       pltpu.VMEM_SHARED((NROW, W2), jnp.float32)],
    )
    def k(msg_hbm, i_hbm, z_hbm, out_hbm, idx_v, rows_v, shared):
        cid = lax.axis_index("c")
        sid = lax.axis_index("s")
        rb = sid * rpt
        pltpu.sync_copy(z_hbm.at[pl.ds(rb, rpt)], shared.at[pl.ds(rb, rpt)])
        plsc.subcore_barrier()

        ibase = cid * (ncp * GCH)

        def one(g):
            pltpu.sync_copy(i_hbm.at[pl.ds(ibase + g * GCH, GCH)], idx_v)
            pltpu.sync_copy(
                msg_hbm.at[pl.ds(jnp.minimum(g, nchunks - 1) * GCH, GCH)],
                rows_v)
            pltpu.sync_copy(rows_v, shared.at[idx_v], add=True)

        def f(c, _):
            one(c * 16 + sid)
            return 0
        lax.fori_loop(0, ncp // 16, f, 0)

        plsc.subcore_barrier()
        pltpu.sync_copy(shared.at[pl.ds(rb, rpt)],
                        out_hbm.at[cid, pl.ds(rb, rpt)])

    return k(msg, iidx, zeros)


def _sc_segmin(vals, idx, nseg, ch=2000):
    nv = vals.shape[0]
    nchunks = nv // ch
    spt = nseg // NW

    @functools.partial(
        pl.kernel,
        out_type=jax.ShapeDtypeStruct((nseg,), jnp.float32),
        mesh=plsc.VectorSubcoreMesh(**_SC_MESH),
        scratch_types=[pltpu.VMEM((ch,), jnp.float32),
                       pltpu.VMEM((ch,), jnp.int32),
                       pltpu.VMEM((spt,), jnp.float32)],
    )
    def k(vals_hbm, idx_hbm, out_hbm, vals_v, idx_v, mloc):
        wid = _wid()
        seg_base = wid * spt
        inf = jnp.float32(jnp.inf)

        def init(q, _):
            mloc[pl.ds(q * 16, 16)] = jnp.full((16,), inf, jnp.float32)
            return 0
        lax.fori_loop(0, spt // 16, init, 0)

        def chunk(gi, _):
            pltpu.sync_copy(vals_hbm.at[pl.ds(gi * ch, ch)], vals_v)
            pltpu.sync_copy(idx_hbm.at[pl.ds(gi * ch, ch)], idx_v)
            _scan_chunk(vals_v, idx_v, mloc, seg_base, spt, ch, False)
            return 0
        lax.fori_loop(0, nchunks, chunk, 0)

        def fin(q, _):
            m = mloc[pl.ds(q * 16, 16)]
            mloc[pl.ds(q * 16, 16)] = jnp.where(m == inf, 0.0, m)
            return 0
        lax.fori_loop(0, spt // 16, fin, 0)
        pltpu.sync_copy(mloc, out_hbm.at[pl.ds(seg_base, spt)])

    return k(vals, idx)


def _sc_segmax_ex(alpha, idx, n_pad, ch=2000):
    ne = alpha.shape[0]
    nchunks = ne // ch
    spt = n_pad // 16

    @functools.partial(
        pl.kernel,
        out_type=jax.ShapeDtypeStruct((ne,), jnp.float32),
        mesh=plsc.VectorSubcoreMesh(**_SC_MESH),
        scratch_types=[pltpu.VMEM((ch,), jnp.float32),
                       pltpu.VMEM((ch,), jnp.int32),
                       pltpu.VMEM((spt,), jnp.float32),
                       pltpu.VMEM((n_pad,), jnp.float32),
                       pltpu.VMEM((ch,), jnp.float32),
                       pltpu.VMEM_SHARED((n_pad,), jnp.float32)],
    )
    def k(a_hbm, i_hbm, ex_hbm, vals_v, idx_v, mloc, amax_v, ex_v, shared):
        cid = lax.axis_index("c")
        sid = lax.axis_index("s")
        wid = sid * 2 + cid
        seg_base = sid * spt
        inf = jnp.float32(jnp.inf)

        def init(q, _):
            mloc[pl.ds(q * 16, 16)] = jnp.full((16,), inf, jnp.float32)
            return 0
        lax.fori_loop(0, spt // 16, init, 0)

        def chunk(gi, _):
            pltpu.sync_copy(a_hbm.at[pl.ds(gi * ch, ch)], vals_v)
            pltpu.sync_copy(i_hbm.at[pl.ds(gi * ch, ch)], idx_v)
            _scan_chunk(vals_v, idx_v, mloc, seg_base, spt, ch, True)
            return 0
        lax.fori_loop(0, nchunks, chunk, 0)

        def fin(q, _):
            m = mloc[pl.ds(q * 16, 16)]
            mloc[pl.ds(q * 16, 16)] = jnp.where(m == inf, 0.0, -m)
            return 0
        lax.fori_loop(0, spt // 16, fin, 0)
        pltpu.sync_copy(mloc, shared.at[pl.ds(seg_base, spt)])
        plsc.subcore_barrier()
        pltpu.sync_copy(shared, amax_v)

        def one(g):
            pltpu.sync_copy(a_hbm.at[pl.ds(g * ch, ch)], vals_v)
            pltpu.sync_copy(i_hbm.at[pl.ds(g * ch, ch)], idx_v)

            def vb(q, _):
                a = vals_v[pl.ds(q * 16, 16)]
                e = idx_v[pl.ds(q * 16, 16)]
                m = plsc.load_gather(amax_v, [e])
                ex_v[pl.ds(q * 16, 16)] = jnp.exp(a - m)
                return 0
            lax.fori_loop(0, ch // 16, vb, 0)
            pltpu.sync_copy(ex_v, ex_hbm.at[pl.ds(g * ch, ch)])
        _rr_loop(nchunks, wid, one)

    return k(alpha, idx)


def _theta_body(cross_ref, dot_ref, out_ref):
    c = cross_ref[...]
    d = dot_ref[...]
    th = jnp.arctan2(jnp.abs(c), d)
    flag = jnp.where(c > 0, 1.0, -1.0)
    out_ref[...] = th * flag


def _geo_body(th_ref, d2_ref, wd1_ref, bd1_ref, wd2_ref, bd2_ref,
              wt1_ref, bt1_ref, wt2_ref, bt2_ref, wg_ref, bg_ref,
              gp0_ref, gp1_ref, gp2_ref):
    th = th_ref[...]
    lane = lax.broadcasted_iota(jnp.int32, (1, H), 1)
    delta = 2.0 * PI / (NG - 1)
    offs = -PI + lane.astype(jnp.float32) * delta
    coeff = -0.5 / (delta * delta)
    texp = jnp.exp(coeff * (th - offs) ** 2)
    texp = jnp.where(lane < NG, texp, 0.0)
    ge2 = jax.nn.relu(jnp.dot(texp, wt1_ref[...]) + bt1_ref[...])
    ge2 = jax.nn.relu(jnp.dot(ge2, wt2_ref[...]) + bt2_ref[...])

    d2 = d2_ref[...]
    invd = jnp.where(d2 == 0.0, 1e10, lax.rsqrt(d2))
    ge1 = jax.nn.relu(invd * wd1_ref[...] + bd1_ref[...])
    ge1 = jax.nn.relu(jnp.dot(ge1, wd2_ref[...]) + bd2_ref[...])

    for b, ref in enumerate((gp0_ref, gp1_ref, gp2_ref)):
        ref[...] = (jnp.dot(ge1, wg_ref[b, :H, :])
                    + jnp.dot(ge2, wg_ref[b, H:, :]) + bg_ref[b])


def _blk_a_body(nfi_ref, nfj_ref, gp_ref, wi_ref, wj_ref, w2_ref, b2_ref,
                w3_ref, b3_ref, w4_ref, b4_ref, att_ref, alpha_ref):
    h = jax.nn.relu(jnp.dot(nfi_ref[...], wi_ref[...])
                    + jnp.dot(nfj_ref[...], wj_ref[...]) + gp_ref[...])
    h = jax.nn.relu(jnp.dot(h, w2_ref[...]) + b2_ref[...])
    h = jax.nn.relu(jnp.dot(h, w3_ref[...]) + b3_ref[...])
    h = jax.nn.relu(jnp.dot(h, w4_ref[...]) + b4_ref[...])
    t = h * att_ref[...]
    t = jnp.where(t >= 0.0, t, 0.01 * t)
    alpha_ref[...] = jnp.sum(t, axis=1, keepdims=True)


def _blk_b_body(nfj_ref, alpha_ref, amaxe_ref, out_ref):
    ex = jnp.exp(alpha_ref[...] - amaxe_ref[...])
    te = ex.shape[0]
    out_ref[...] = jnp.concatenate(
        [nfj_ref[...] * ex, ex, jnp.zeros((te, H - 1), jnp.float32)], axis=1)


def _fin_body(acc_ref, nf_ref, src_ref, out_ref):
    acc = acc_ref[...]
    numer = acc[:, :H]
    denom = acc[:, H:H + 1]
    out_ref[...] = jnp.where(src_ref[...] > 0.0, nf_ref[...],
                             numer / (denom + 1e-16))


def _theta_call(cross2d, dot2d):
    nr = cross2d.shape[0]
    br = 1000
    spec = pl.BlockSpec((br, H), lambda i: (i, 0))
    return pl.pallas_call(
        _theta_body,
        grid=(nr // br,),
        in_specs=[spec, spec],
        out_specs=spec,
        out_shape=jax.ShapeDtypeStruct((nr, H), jnp.float32),
    )(cross2d, dot2d)


def _geo_call(thetas1, d2, wp):
    ne = thetas1.shape[0]
    te = 2000
    col = pl.BlockSpec((te, 1), lambda i: (i, 0))
    out = pl.BlockSpec((te, H), lambda i: (i, 0))
    full = lambda a: pl.BlockSpec(a.shape, lambda i: (0,) * a.ndim)
    w_list = [wp['wd1'], wp['bd1'], wp['wd2'], wp['bd2'],
              wp['wt1'], wp['bt1'], wp['wt2'], wp['bt2'],
              wp['wg'], wp['bg']]
    return pl.pallas_call(
        _geo_body,
        grid=(ne // te,),
        in_specs=[col, col] + [full(w) for w in w_list],
        out_specs=[out, out, out],
        out_shape=[jax.ShapeDtypeStruct((ne, H), jnp.float32)] * 3,
    )(thetas1, d2, *w_list)


def _blk_a_call(nfi_g, nfj_g, gp, bw):
    ne = nfi_g.shape[0]
    te = 2000
    mat = pl.BlockSpec((te, H), lambda i: (i, 0))
    col = pl.BlockSpec((te, 1), lambda i: (i, 0))
    full = lambda a: pl.BlockSpec(a.shape, lambda i: (0,) * a.ndim)
    w_list = [bw['wi'], bw['wj'], bw['w2'], bw['b2'],
              bw['w3'], bw['b3'], bw['w4'], bw['b4'], bw['att']]
    return pl.pallas_call(
        _blk_a_body,
        grid=(ne // te,),
        in_specs=[mat, mat, mat] + [full(w) for w in w_list],
        out_specs=col,
        out_shape=jax.ShapeDtypeStruct((ne, 1), jnp.float32),
    )(nfi_g, nfj_g, gp, *w_list)


def _blk_b_call(nfj_g, alpha, amax_e):
    ne = nfj_g.shape[0]
    te = 2000
    mat = pl.BlockSpec((te, H), lambda i: (i, 0))
    col = pl.BlockSpec((te, 1), lambda i: (i, 0))
    out = pl.BlockSpec((te, MSGW), lambda i: (i, 0))
    return pl.pallas_call(
        _blk_b_body,
        grid=(ne // te,),
        in_specs=[mat, col, col],
        out_specs=pl.BlockSpec((te, W2), lambda i: (i, 0)),
        out_shape=jax.ShapeDtypeStruct((ne, W2), jnp.float32),
    )(nfj_g, alpha, amax_e)


def _fin_call(acc, nf, srcmask):
    n = nf.shape[0]
    tn = 2000
    return pl.pallas_call(
        _fin_body,
        grid=(n // tn,),
        in_specs=[pl.BlockSpec((tn, W2), lambda i: (i, 0)),
                  pl.BlockSpec((tn, H), lambda i: (i, 0)),
                  pl.BlockSpec((tn, 1), lambda i: (i, 0))],
        out_specs=pl.BlockSpec((tn, H), lambda i: (i, 0)),
        out_shape=jax.ShapeDtypeStruct((n, H), jnp.float32),
    )(acc, nf, srcmask)


def kernel(coords, edge_index, edge_index_2rd, edx_1st, edx_2nd, E,
           is_source, edge_rep, params):
    n = coords.shape[0]
    ne = edge_index.shape[1]
    nt = edge_index_2rd.shape[1]

    (wd1, bd1), (wd2, bd2) = params['dist']
    (wt1, bt1), (wt2, bt2) = params['theta']
    wt1p = jnp.zeros((H, H), jnp.float32).at[:NG, :].set(wt1.T)
    wp = dict(wd1=wd1.T, bd1=bd1[None, :], wd2=wd2.T, bd2=bd2[None, :],
              wt1=wt1p, bt1=bt1[None, :], wt2=wt2.T, bt2=bt2[None, :])
    wgs, bgs, bws = [], [], []
    for blk in params['blocks']:
        (w1, b1) = blk['lins'][0]
        wgs.append(w1[:, 2 * H:].T)
        bgs.append(b1[None, :])
        bw = dict(wi=w1[:, :H].T, wj=w1[:, H:2 * H].T, att=blk['att'])
        for li, (w, b) in enumerate(blk['lins'][1:], start=2):
            bw[f'w{li}'] = w.T
            bw[f'b{li}'] = b[None, :]
        bws.append(bw)
    wp['wg'] = jnp.stack(wgs)
    wp['bg'] = jnp.stack(bgs)

    i2, j2, k2 = edge_index_2rd
    g = coords
    v1 = g[j2] - g[i2]
    v2 = g[k2] - g[j2]
    cross = v1[:, 0] * v2[:, 1] - v1[:, 1] * v2[:, 0]
    dot = (v1 * v2).sum(axis=-1)
    nr = nt // H
    vals = _theta_call(cross.reshape(nr, H), dot.reshape(nr, H)).reshape(nt)
    if _USE_SC_SEGMIN:
        thetas1 = _sc_segmin(vals, edx_2nd.astype(jnp.int32), ne)
    else:
        seg_min = jax.ops.segment_min(vals, edx_2nd, num_segments=ne)
        thetas1 = jnp.where(jnp.isfinite(seg_min), seg_min, 0.0)

    de = g[edge_index[0]] - g[edge_index[1]]
    d2 = (de * de).sum(axis=-1)

    gp0, gp1, gp2 = _geo_call(thetas1[:, None], d2[:, None], wp)

    srcmask = is_source.astype(jnp.float32)[:, None]
    jidx = edge_index[0].astype(jnp.int32)
    iidx = edge_index[1].astype(jnp.int32)
    zeros = jnp.zeros((NROW, W2), jnp.float32)
    nc = ne // GCH
    pad = (-(-nc // 16) * 16) * GCH - ne
    idx2 = []
    for c in (0, 1):
        loc = iidx - c * HALF
        loc = jnp.where((loc >= 0) & (loc < HALF), loc, HALF)
        idx2.append(jnp.pad(loc, (0, pad), constant_values=HALF))
    iidx2 = jnp.concatenate(idx2)
    nf = E
    for b, gp in enumerate((gp0, gp1, gp2)):
        nfi_g, nfj_g = _sc_gather2(nf, iidx, jidx)
        alpha = _blk_a_call(nfi_g, nfj_g, gp, bws[b])
        amax = jax.ops.segment_max(alpha[:, 0], iidx, num_segments=n)
        amax = jnp.where(jnp.isfinite(amax), amax, 0.0)
        msg = _blk_b_call(nfj_g, alpha, amax[iidx][:, None])
        acc = jax.ops.segment_sum(msg, iidx, num_segments=n)
        nf = _fin_call(acc, nf, srcmask)
    return nf
